# Initial kernel scaffold; baseline (speedup 1.0000x reference)
import jax
import jax.numpy as jnp
from jax.experimental import pallas as pl


def kernel(ray_o, ray_d, points):
    raise NotImplementedError("write your pallas kernel here")


# speedup vs baseline: 13.1241x; 13.1241x over previous
import functools

import jax
import jax.numpy as jnp
from jax import lax
from jax.experimental import pallas as pl
from jax.experimental.pallas import tpu as pltpu
from jax.experimental.pallas import tpu_sc as plsc

R = 1024
N = 65536
K = 32
NC, NS = 2, 16
NW = NC * NS
RPW = R // NW
CHM = 2048
HALF = N // 2
UNITS = 2 * RPW
VGRP = HALF // 32
GCH = 32768
EPS = 1e-6
INF = float("inf")
IMAX = 2**31 - 1


def _mask_body(d_ref, o_ref, ptx_ref, out_ref):
    vx = ptx_ref[0:1, :] - o_ref[0, 0]
    vy = ptx_ref[1:2, :] - o_ref[0, 1]
    vz = ptx_ref[2:3, :] - o_ref[0, 2]
    vcat = jnp.concatenate([vx, vy, vz], axis=0)
    sq = vcat * vcat
    dsq = jnp.sum(sq, axis=0, keepdims=True)
    db = d_ref[...].astype(jnp.bfloat16)
    vb = vcat.astype(jnp.bfloat16)
    proj = jnp.dot(db, vb, preferred_element_type=jnp.float32)
    perp = jnp.maximum(dsq - proj * proj, 0.0)
    out_ref[...] = jnp.where(proj > 0.0, perp, jnp.inf)


def _masked_tc(d, ray_o, ptx):
    return pl.pallas_call(
        _mask_body,
        grid=(N // CHM,),
        in_specs=[pl.BlockSpec((R, 3), lambda i: (0, 0)),
                  pl.BlockSpec((1, 3), lambda i: (0, 0)),
                  pl.BlockSpec((3, CHM), lambda i: (0, i))],
        out_specs=pl.BlockSpec((R, CHM), lambda i: (0, i)),
        out_shape=jax.ShapeDtypeStruct((R, N), jnp.float32),
    )(d, ray_o, ptx)


def _gat(x, idx):
    return x.at[idx].get(mode="promise_in_bounds")


def _splat_last(x):
    return _gat(x, jnp.full((16,), 15, jnp.int32))


def _vmin_splat(x):
    lane = lax.iota(jnp.int32, 16)
    for m in (8, 4, 2, 1):
        x = jnp.minimum(x, _gat(x, lane ^ m))
    return x


def _merge(ckm, civ, t0k, t0i, t1k, t1i):
    ck, ci = plsc.sort_key_val(ckm, civ)
    rk = lax.rev(ck, (0,))
    ri = lax.rev(ci, (0,))
    c1 = rk < t1k
    mk0 = jnp.where(c1, rk, t1k)
    mi0 = jnp.where(c1, ri, t1i)
    mk, mi = plsc.sort_key_val(mk0, mi0)
    rmk = lax.rev(mk, (0,))
    rmi = lax.rev(mi, (0,))
    c2 = rmk < t0k
    lok = jnp.where(c2, rmk, t0k)
    loi = jnp.where(c2, rmi, t0i)
    hik = jnp.where(c2, t0k, rmk)
    hii = jnp.where(c2, t0i, rmi)
    nt0k, nt0i = plsc.sort_key_val(lok, loi)
    nt1k, nt1i = plsc.sort_key_val(hik, hii)
    return nt0k, nt0i, nt1k, nt1i, _splat_last(nt1k)


def _worker_id():
    return lax.axis_index("s") * NC + lax.axis_index("c")


def _scan_unit(buf, colbase, state):
    def ibody(i, carry):
        t0k, t0i, t1k, t1i, thr = carry
        off = i * 32
        k0v = buf[pl.ds(off, 16)]
        k1v = buf[pl.ds(off + 16, 16)]
        ok0 = k0v < thr
        ok1 = k1v < thr

        def do(args):
            ok0, ok1, k0v, k1v, off, t0k, t0i, t1k, t1i, thr = args

            def do0(a):
                ok0, k0v, off, t0k, t0i, t1k, t1i, _thr = a
                ckm = jnp.where(ok0, k0v, INF)
                civ = lax.iota(jnp.int32, 16) + (colbase + off)
                return _merge(ckm, civ, t0k, t0i, t1k, t1i)

            def skip0(a):
                _ok0, _k0v, _off, t0k, t0i, t1k, t1i, thr = a
                return t0k, t0i, t1k, t1i, thr

            t0k, t0i, t1k, t1i, thr = lax.cond(
                jnp.any(ok0), do0, skip0,
                (ok0, k0v, off, t0k, t0i, t1k, t1i, thr))

            def do1(a):
                ok1, k1v, off, t0k, t0i, t1k, t1i, thr = a
                ok1 = ok1 & (k1v < thr)
                ckm = jnp.where(ok1, k1v, INF)
                civ = lax.iota(jnp.int32, 16) + (colbase + off + 16)
                return _merge(ckm, civ, t0k, t0i, t1k, t1i)

            def skip1(a):
                _ok1, _k1v, _off, t0k, t0i, t1k, t1i, thr = a
                return t0k, t0i, t1k, t1i, thr

            return lax.cond(jnp.any(ok1), do1, skip1,
                            (ok1, k1v, off, t0k, t0i, t1k, t1i, thr))

        def skip(args):
            return args[5:]

        return lax.cond(jnp.any(ok0) | jnp.any(ok1), do, skip,
                        (ok0, ok1, k0v, k1v, off, t0k, t0i, t1k, t1i, thr))

    return lax.fori_loop(0, VGRP, ibody, state)


def _sc_body(keys_hbm, px_hbm, py_hbm, pz_hbm,
             idx_hbm, perp_hbm, sx_hbm, sy_hbm, sz_hbm,
             bufA, bufB, bufC, stk, sti, okk, oii,
             sxv, syv, szv, semA, semB):
    wid = _worker_id()
    ray_base = wid * RPW
    out_base = wid * (RPW * K)

    def init_body(j, _):
        stk[pl.ds(j * 16, 16)] = jnp.full((16,), INF, jnp.float32)
        sti[pl.ds(j * 16, 16)] = jnp.zeros((16,), jnp.int32)
        return 0
    lax.fori_loop(0, 2 * RPW, init_body, 0)

    def unit_src(u):
        base = (ray_base + (u // 2)) * N + (u % 2) * HALF
        return keys_hbm.at[pl.ds(base, HALF)]

    def load_state(r):
        s0 = r * 32
        return (stk[pl.ds(s0, 16)], sti[pl.ds(s0, 16)],
                stk[pl.ds(s0 + 16, 16)], sti[pl.ds(s0 + 16, 16)])

    def store_state(r, st):
        t0k, t0i, t1k, t1i, _thr = st
        s0 = r * 32
        stk[pl.ds(s0, 16)] = t0k
        sti[pl.ds(s0, 16)] = t0i
        stk[pl.ds(s0 + 16, 16)] = t1k
        sti[pl.ds(s0 + 16, 16)] = t1i
        return 0

    def process(u, buf):
        r = u // 2
        t0k, t0i, t1k, t1i = load_state(r)
        st = (t0k, t0i, t1k, t1i, _splat_last(t1k))
        st = _scan_unit(buf, (u % 2) * HALF, st)
        return store_state(r, st)

    pltpu.async_copy(unit_src(0), bufA, semA)

    def pair_body(p, _):
        u0 = 2 * p
        pltpu.async_copy(unit_src(u0 + 1), bufB, semB)
        pltpu.make_async_copy(unit_src(u0), bufA, semA).wait()
        process(u0, bufA)

        @pl.when(p + 1 < UNITS // 2)
        def _():
            pltpu.async_copy(unit_src(u0 + 2), bufA, semA)
        pltpu.make_async_copy(unit_src(u0 + 1), bufB, semB).wait()
        process(u0 + 1, bufB)
        return 0
    lax.fori_loop(0, UNITS // 2, pair_body, 0)

    lane = lax.iota(jnp.int32, 16)

    def fbody(r, _):
        s0 = r * 32
        k0 = stk[pl.ds(s0, 16)]
        k1 = stk[pl.ds(s0 + 16, 16)]
        i0 = sti[pl.ds(s0, 16)]
        i1 = sti[pl.ds(s0 + 16, 16)]
        ok0 = jnp.zeros((16,), jnp.float32)
        ok1 = jnp.zeros((16,), jnp.float32)
        oi0 = jnp.zeros((16,), jnp.int32)
        oi1 = jnp.zeros((16,), jnp.int32)
        for j in range(K):
            mn = _vmin_splat(jnp.minimum(k0, k1))
            c0 = k0 == mn
            c1 = k1 == mn
            mi = _vmin_splat(jnp.minimum(jnp.where(c0, i0, IMAX),
                                         jnp.where(c1, i1, IMAX)))
            if j < 16:
                sel = lane == j
                ok0 = jnp.where(sel, mn, ok0)
                oi0 = jnp.where(sel, mi, oi0)
            else:
                sel = lane == (j - 16)
                ok1 = jnp.where(sel, mn, ok1)
                oi1 = jnp.where(sel, mi, oi1)
            r0 = c0 & (i0 == mi)
            r1 = c1 & (i1 == mi)
            k0 = jnp.where(r0, INF, k0)
            i0 = jnp.where(r0, IMAX, i0)
            k1 = jnp.where(r1, INF, k1)
            i1 = jnp.where(r1, IMAX, i1)
        okk[pl.ds(s0, 16)] = ok0
        okk[pl.ds(s0 + 16, 16)] = ok1
        oii[pl.ds(s0, 16)] = oi0
        oii[pl.ds(s0 + 16, 16)] = oi1
        return 0
    lax.fori_loop(0, RPW, fbody, 0)

    for c in range(N // GCH):
        sl = pl.ds(c * GCH, GCH)
        pltpu.sync_copy(px_hbm.at[sl], bufA)
        pltpu.sync_copy(py_hbm.at[sl], bufB)
        pltpu.sync_copy(pz_hbm.at[sl], bufC)
        cbase = c * GCH

        def gbody(j, _):
            sl16 = pl.ds(j * 16, 16)
            idxv = oii[sl16]
            inr = (idxv >= cbase) & (idxv < cbase + GCH)
            li = jnp.where(inr, idxv - cbase, 0)
            gx = plsc.load_gather(bufA, [li], mask=inr)
            gy = plsc.load_gather(bufB, [li], mask=inr)
            gz = plsc.load_gather(bufC, [li], mask=inr)
            sxv[sl16] = jnp.where(inr, gx, sxv[sl16])
            syv[sl16] = jnp.where(inr, gy, syv[sl16])
            szv[sl16] = jnp.where(inr, gz, szv[sl16])
            return 0
        lax.fori_loop(0, 2 * RPW, gbody, 0)

    pltpu.sync_copy(oii, idx_hbm.at[pl.ds(out_base, RPW * K)])
    pltpu.sync_copy(okk, perp_hbm.at[pl.ds(out_base, RPW * K)])
    pltpu.sync_copy(sxv, sx_hbm.at[pl.ds(out_base, RPW * K)])
    pltpu.sync_copy(syv, sy_hbm.at[pl.ds(out_base, RPW * K)])
    pltpu.sync_copy(szv, sz_hbm.at[pl.ds(out_base, RPW * K)])


def _sc_search(keys, px, py, pz):
    mesh = plsc.VectorSubcoreMesh(
        core_axis_name="c", subcore_axis_name="s",
        num_cores=NC, num_subcores=NS)
    f32 = jnp.float32
    kern = functools.partial(
        pl.kernel,
        out_type=[jax.ShapeDtypeStruct((R * K,), jnp.int32),
                  jax.ShapeDtypeStruct((R * K,), f32),
                  jax.ShapeDtypeStruct((R * K,), f32),
                  jax.ShapeDtypeStruct((R * K,), f32),
                  jax.ShapeDtypeStruct((R * K,), f32)],
        mesh=mesh,
        compiler_params=pltpu.CompilerParams(needs_layout_passes=False),
        scratch_types=(
            [pltpu.VMEM((HALF,), f32)] * 3
            + [pltpu.VMEM((RPW * K,), f32),
               pltpu.VMEM((RPW * K,), jnp.int32),
               pltpu.VMEM((RPW * K,), f32),
               pltpu.VMEM((RPW * K,), jnp.int32),
               pltpu.VMEM((RPW * K,), f32),
               pltpu.VMEM((RPW * K,), f32),
               pltpu.VMEM((RPW * K,), f32),
               pltpu.SemaphoreType.DMA,
               pltpu.SemaphoreType.DMA]))(_sc_body)
    return kern(keys, px, py, pz)


def _feat_body(sx_ref, sy_ref, sz_ref, perp_ref, d_ref, o_ref,
               dist_ref, projd_ref, az_ref, pitch_ref, sky_ref):
    ox = o_ref[0, 0]
    oy = o_ref[0, 1]
    oz = o_ref[0, 2]
    vx = sx_ref[...] - ox
    vy = sy_ref[...] - oy
    vz = sz_ref[...] - oz
    dist = jnp.sqrt((vx * vx + vy * vy + vz * vz) + EPS)
    dx = d_ref[:, 0:1]
    dy = d_ref[:, 1:2]
    dz = d_ref[:, 2:3]
    dist_ref[...] = dist

    def b16(x):
        return x.astype(jnp.bfloat16).astype(jnp.float32)
    projd_ref[...] = (b16(dx) * b16(vx) + b16(dy) * b16(vy)
                      + b16(dz) * b16(vz))
    az_ref[...] = jnp.arctan2(vy, vx)
    ct = jnp.clip(vz / dist, -1.0 + EPS, 1.0 - EPS)
    st = jnp.sqrt(jnp.maximum((1.0 - ct) * (1.0 + ct), 0.0))
    pitch_ref[...] = jnp.arctan2(st, ct)
    sky_ref[...] = perp_ref[:, 0:1] > 1.0


def _features(sx, sy, sz, perp, d, ray_o):
    f32 = jnp.float32
    return pl.pallas_call(
        _feat_body,
        out_shape=[jax.ShapeDtypeStruct((R, K), f32),
                   jax.ShapeDtypeStruct((R, K), f32),
                   jax.ShapeDtypeStruct((R, K), f32),
                   jax.ShapeDtypeStruct((R, K), f32),
                   jax.ShapeDtypeStruct((R, 1), jnp.bool_)],
    )(sx, sy, sz, perp, d, ray_o)


def kernel(ray_o, ray_d, points):
    d = ray_d / (jnp.linalg.norm(ray_d, axis=-1, keepdims=True) + EPS)
    ray_info = jnp.concatenate(
        [jnp.broadcast_to(ray_o, (R, 3)), d], axis=-1)
    ptx = points.T
    masked = _masked_tc(d, ray_o, ptx)
    keys = jnp.reshape(masked, (R * N,))
    idxf, perpf, sxf, syf, szf = _sc_search(keys, ptx[0], ptx[1], ptx[2])
    idx = idxf.reshape(R, K)
    perp = perpf.reshape(R, K)
    sx = sxf.reshape(R, K)
    sy = syf.reshape(R, K)
    sz = szf.reshape(R, K)
    dist, projd, az, pitch, sky = _features(sx, sy, sz, perp, d, ray_o)
    points_info = jnp.stack([sx, sy, sz, dist, projd, az, pitch], axis=-1)
    return (points_info, ray_info, idx[..., None], sky)


# speedup vs baseline: 14.5867x; 1.1114x over previous
import functools

import jax
import jax.numpy as jnp
from jax import lax
from jax.experimental import pallas as pl
from jax.experimental.pallas import tpu as pltpu
from jax.experimental.pallas import tpu_sc as plsc

R = 1024
N = 65536
K = 32
NC, NS = 2, 16
NW = NC * NS
RPW = R // NW
CHM = 2048
HALF = N // 2
UNITS = 2 * RPW
VGRP = HALF // 32
GCH = 32768
EPS = 1e-6
INF = float("inf")
IMAX = 2**31 - 1


def _mask_body(d_ref, o_ref, ptx_ref, out_ref):
    vx = ptx_ref[0:1, :] - o_ref[0, 0]
    vy = ptx_ref[1:2, :] - o_ref[0, 1]
    vz = ptx_ref[2:3, :] - o_ref[0, 2]
    vcat = jnp.concatenate([vx, vy, vz], axis=0)
    sq = vcat * vcat
    dsq = jnp.sum(sq, axis=0, keepdims=True)
    db = d_ref[...].astype(jnp.bfloat16)
    vb = vcat.astype(jnp.bfloat16)
    proj = jnp.dot(db, vb, preferred_element_type=jnp.float32)
    perp = jnp.maximum(dsq - proj * proj, 0.0)
    out_ref[...] = jnp.where(proj > 0.0, perp, jnp.inf)


def _masked_tc(d, ray_o, ptx):
    return pl.pallas_call(
        _mask_body,
        grid=(N // CHM,),
        in_specs=[pl.BlockSpec((R, 3), lambda i: (0, 0)),
                  pl.BlockSpec((1, 3), lambda i: (0, 0)),
                  pl.BlockSpec((3, CHM), lambda i: (0, i))],
        out_specs=pl.BlockSpec((R, CHM), lambda i: (0, i)),
        out_shape=jax.ShapeDtypeStruct((R, N), jnp.float32),
    )(d, ray_o, ptx)


def _gat(x, idx):
    return x.at[idx].get(mode="promise_in_bounds")


def _splat_last(x):
    return _gat(x, jnp.full((16,), 15, jnp.int32))


def _vmin_splat(x):
    lane = lax.iota(jnp.int32, 16)
    for m in (8, 4, 2, 1):
        x = jnp.minimum(x, _gat(x, lane ^ m))
    return x


def _merge(ckm, civ, t0k, t0i, t1k, t1i):
    ck, ci = plsc.sort_key_val(ckm, civ)
    rk = lax.rev(ck, (0,))
    ri = lax.rev(ci, (0,))
    c1 = rk < t1k
    mk0 = jnp.where(c1, rk, t1k)
    mi0 = jnp.where(c1, ri, t1i)
    mk, mi = plsc.sort_key_val(mk0, mi0)
    rmk = lax.rev(mk, (0,))
    rmi = lax.rev(mi, (0,))
    c2 = rmk < t0k
    lok = jnp.where(c2, rmk, t0k)
    loi = jnp.where(c2, rmi, t0i)
    hik = jnp.where(c2, t0k, rmk)
    hii = jnp.where(c2, t0i, rmi)
    nt0k, nt0i = plsc.sort_key_val(lok, loi)
    nt1k, nt1i = plsc.sort_key_val(hik, hii)
    return nt0k, nt0i, nt1k, nt1i, _splat_last(nt1k)


def _worker_id():
    return lax.axis_index("s") * NC + lax.axis_index("c")


def _scan_unit(buf, colbase, state):
    def ibody(i, carry):
        t0k, t0i, t1k, t1i, thr = carry
        off = i * 32
        k0v = buf[pl.ds(off, 16)]
        k1v = buf[pl.ds(off + 16, 16)]
        ok0 = k0v < thr
        ok1 = k1v < thr

        def do(args):
            ok0, ok1, k0v, k1v, off, t0k, t0i, t1k, t1i, thr = args

            def do0(a):
                ok0, k0v, off, t0k, t0i, t1k, t1i, _thr = a
                ckm = jnp.where(ok0, k0v, INF)
                civ = lax.iota(jnp.int32, 16) + (colbase + off)
                return _merge(ckm, civ, t0k, t0i, t1k, t1i)

            def skip0(a):
                _ok0, _k0v, _off, t0k, t0i, t1k, t1i, thr = a
                return t0k, t0i, t1k, t1i, thr

            t0k, t0i, t1k, t1i, thr = lax.cond(
                plsc.all_reduce_population_count(ok0)[0] > 0, do0, skip0,
                (ok0, k0v, off, t0k, t0i, t1k, t1i, thr))

            def do1(a):
                ok1, k1v, off, t0k, t0i, t1k, t1i, thr = a
                ok1 = ok1 & (k1v < thr)
                ckm = jnp.where(ok1, k1v, INF)
                civ = lax.iota(jnp.int32, 16) + (colbase + off + 16)
                return _merge(ckm, civ, t0k, t0i, t1k, t1i)

            def skip1(a):
                _ok1, _k1v, _off, t0k, t0i, t1k, t1i, thr = a
                return t0k, t0i, t1k, t1i, thr

            return lax.cond(
                plsc.all_reduce_population_count(ok1)[0] > 0, do1, skip1,
                (ok1, k1v, off, t0k, t0i, t1k, t1i, thr))

        def skip(args):
            return args[5:]

        anycnt = plsc.all_reduce_population_count(ok0 | ok1)
        return lax.cond(anycnt[0] > 0, do, skip,
                        (ok0, ok1, k0v, k1v, off, t0k, t0i, t1k, t1i, thr))

    return lax.fori_loop(0, VGRP, ibody, state)


def _sc_body(keys_hbm, px_hbm, py_hbm, pz_hbm,
             idx_hbm, perp_hbm, sx_hbm, sy_hbm, sz_hbm,
             bufA, bufB, bufC, stk, sti, okk, oii,
             sxv, syv, szv, semA, semB):
    wid = _worker_id()
    ray_base = wid * RPW
    out_base = wid * (RPW * K)

    def init_body(j, _):
        stk[pl.ds(j * 16, 16)] = jnp.full((16,), INF, jnp.float32)
        sti[pl.ds(j * 16, 16)] = jnp.zeros((16,), jnp.int32)
        return 0
    lax.fori_loop(0, 2 * RPW, init_body, 0)

    def unit_src(u):
        base = (ray_base + (u // 2)) * N + (u % 2) * HALF
        return keys_hbm.at[pl.ds(base, HALF)]

    def load_state(r):
        s0 = r * 32
        return (stk[pl.ds(s0, 16)], sti[pl.ds(s0, 16)],
                stk[pl.ds(s0 + 16, 16)], sti[pl.ds(s0 + 16, 16)])

    def store_state(r, st):
        t0k, t0i, t1k, t1i, _thr = st
        s0 = r * 32
        stk[pl.ds(s0, 16)] = t0k
        sti[pl.ds(s0, 16)] = t0i
        stk[pl.ds(s0 + 16, 16)] = t1k
        sti[pl.ds(s0 + 16, 16)] = t1i
        return 0

    def process(u, buf):
        r = u // 2
        t0k, t0i, t1k, t1i = load_state(r)
        st = (t0k, t0i, t1k, t1i, _splat_last(t1k))
        st = _scan_unit(buf, (u % 2) * HALF, st)
        return store_state(r, st)

    pltpu.async_copy(unit_src(0), bufA, semA)

    def pair_body(p, _):
        u0 = 2 * p
        pltpu.async_copy(unit_src(u0 + 1), bufB, semB)
        pltpu.make_async_copy(unit_src(u0), bufA, semA).wait()
        process(u0, bufA)

        @pl.when(p + 1 < UNITS // 2)
        def _():
            pltpu.async_copy(unit_src(u0 + 2), bufA, semA)
        pltpu.make_async_copy(unit_src(u0 + 1), bufB, semB).wait()
        process(u0 + 1, bufB)
        return 0
    lax.fori_loop(0, UNITS // 2, pair_body, 0)

    lane = lax.iota(jnp.int32, 16)

    def fbody(r, _):
        s0 = r * 32
        k0 = stk[pl.ds(s0, 16)]
        k1 = stk[pl.ds(s0 + 16, 16)]
        i0 = sti[pl.ds(s0, 16)]
        i1 = sti[pl.ds(s0 + 16, 16)]
        ok0 = jnp.zeros((16,), jnp.float32)
        ok1 = jnp.zeros((16,), jnp.float32)
        oi0 = jnp.zeros((16,), jnp.int32)
        oi1 = jnp.zeros((16,), jnp.int32)
        for j in range(K):
            mn = _vmin_splat(jnp.minimum(k0, k1))
            c0 = k0 == mn
            c1 = k1 == mn
            mi = _vmin_splat(jnp.minimum(jnp.where(c0, i0, IMAX),
                                         jnp.where(c1, i1, IMAX)))
            if j < 16:
                sel = lane == j
                ok0 = jnp.where(sel, mn, ok0)
                oi0 = jnp.where(sel, mi, oi0)
            else:
                sel = lane == (j - 16)
                ok1 = jnp.where(sel, mn, ok1)
                oi1 = jnp.where(sel, mi, oi1)
            r0 = c0 & (i0 == mi)
            r1 = c1 & (i1 == mi)
            k0 = jnp.where(r0, INF, k0)
            i0 = jnp.where(r0, IMAX, i0)
            k1 = jnp.where(r1, INF, k1)
            i1 = jnp.where(r1, IMAX, i1)
        okk[pl.ds(s0, 16)] = ok0
        okk[pl.ds(s0 + 16, 16)] = ok1
        oii[pl.ds(s0, 16)] = oi0
        oii[pl.ds(s0 + 16, 16)] = oi1
        return 0
    lax.fori_loop(0, RPW, fbody, 0)

    for c in range(N // GCH):
        sl = pl.ds(c * GCH, GCH)
        pltpu.sync_copy(px_hbm.at[sl], bufA)
        pltpu.sync_copy(py_hbm.at[sl], bufB)
        pltpu.sync_copy(pz_hbm.at[sl], bufC)
        cbase = c * GCH

        def gbody(j, _):
            sl16 = pl.ds(j * 16, 16)
            idxv = oii[sl16]
            inr = (idxv >= cbase) & (idxv < cbase + GCH)
            li = jnp.where(inr, idxv - cbase, 0)
            gx = plsc.load_gather(bufA, [li], mask=inr)
            gy = plsc.load_gather(bufB, [li], mask=inr)
            gz = plsc.load_gather(bufC, [li], mask=inr)
            sxv[sl16] = jnp.where(inr, gx, sxv[sl16])
            syv[sl16] = jnp.where(inr, gy, syv[sl16])
            szv[sl16] = jnp.where(inr, gz, szv[sl16])
            return 0
        lax.fori_loop(0, 2 * RPW, gbody, 0)

    pltpu.sync_copy(oii, idx_hbm.at[pl.ds(out_base, RPW * K)])
    pltpu.sync_copy(okk, perp_hbm.at[pl.ds(out_base, RPW * K)])
    pltpu.sync_copy(sxv, sx_hbm.at[pl.ds(out_base, RPW * K)])
    pltpu.sync_copy(syv, sy_hbm.at[pl.ds(out_base, RPW * K)])
    pltpu.sync_copy(szv, sz_hbm.at[pl.ds(out_base, RPW * K)])


def _sc_search(keys, px, py, pz):
    mesh = plsc.VectorSubcoreMesh(
        core_axis_name="c", subcore_axis_name="s",
        num_cores=NC, num_subcores=NS)
    f32 = jnp.float32
    kern = functools.partial(
        pl.kernel,
        out_type=[jax.ShapeDtypeStruct((R * K,), jnp.int32),
                  jax.ShapeDtypeStruct((R * K,), f32),
                  jax.ShapeDtypeStruct((R * K,), f32),
                  jax.ShapeDtypeStruct((R * K,), f32),
                  jax.ShapeDtypeStruct((R * K,), f32)],
        mesh=mesh,
        compiler_params=pltpu.CompilerParams(needs_layout_passes=False),
        scratch_types=(
            [pltpu.VMEM((HALF,), f32)] * 3
            + [pltpu.VMEM((RPW * K,), f32),
               pltpu.VMEM((RPW * K,), jnp.int32),
               pltpu.VMEM((RPW * K,), f32),
               pltpu.VMEM((RPW * K,), jnp.int32),
               pltpu.VMEM((RPW * K,), f32),
               pltpu.VMEM((RPW * K,), f32),
               pltpu.VMEM((RPW * K,), f32),
               pltpu.SemaphoreType.DMA,
               pltpu.SemaphoreType.DMA]))(_sc_body)
    return kern(keys, px, py, pz)


def _feat_body(sx_ref, sy_ref, sz_ref, perp_ref, d_ref, o_ref,
               dist_ref, projd_ref, az_ref, pitch_ref, sky_ref):
    ox = o_ref[0, 0]
    oy = o_ref[0, 1]
    oz = o_ref[0, 2]
    vx = sx_ref[...] - ox
    vy = sy_ref[...] - oy
    vz = sz_ref[...] - oz
    dist = jnp.sqrt((vx * vx + vy * vy + vz * vz) + EPS)
    dx = d_ref[:, 0:1]
    dy = d_ref[:, 1:2]
    dz = d_ref[:, 2:3]
    dist_ref[...] = dist

    def b16(x):
        return x.astype(jnp.bfloat16).astype(jnp.float32)
    projd_ref[...] = (b16(dx) * b16(vx) + b16(dy) * b16(vy)
                      + b16(dz) * b16(vz))
    az_ref[...] = jnp.arctan2(vy, vx)
    ct = jnp.clip(vz / dist, -1.0 + EPS, 1.0 - EPS)
    st = jnp.sqrt(jnp.maximum((1.0 - ct) * (1.0 + ct), 0.0))
    pitch_ref[...] = jnp.arctan2(st, ct)
    sky_ref[...] = perp_ref[:, 0:1] > 1.0


def _features(sx, sy, sz, perp, d, ray_o):
    f32 = jnp.float32
    return pl.pallas_call(
        _feat_body,
        out_shape=[jax.ShapeDtypeStruct((R, K), f32),
                   jax.ShapeDtypeStruct((R, K), f32),
                   jax.ShapeDtypeStruct((R, K), f32),
                   jax.ShapeDtypeStruct((R, K), f32),
                   jax.ShapeDtypeStruct((R, 1), jnp.bool_)],
    )(sx, sy, sz, perp, d, ray_o)


def kernel(ray_o, ray_d, points):
    d = ray_d / (jnp.linalg.norm(ray_d, axis=-1, keepdims=True) + EPS)
    ray_info = jnp.concatenate(
        [jnp.broadcast_to(ray_o, (R, 3)), d], axis=-1)
    ptx = points.T
    masked = _masked_tc(d, ray_o, ptx)
    keys = jnp.reshape(masked, (R * N,))
    idxf, perpf, sxf, syf, szf = _sc_search(keys, ptx[0], ptx[1], ptx[2])
    idx = idxf.reshape(R, K)
    perp = perpf.reshape(R, K)
    sx = sxf.reshape(R, K)
    sy = syf.reshape(R, K)
    sz = szf.reshape(R, K)
    dist, projd, az, pitch, sky = _features(sx, sy, sz, perp, d, ray_o)
    points_info = jnp.stack([sx, sy, sz, dist, projd, az, pitch], axis=-1)
    return (points_info, ray_info, idx[..., None], sky)


# speedup vs baseline: 17.2212x; 1.1806x over previous
import functools

import jax
import jax.numpy as jnp
from jax import lax
from jax.experimental import pallas as pl
from jax.experimental.pallas import tpu as pltpu
from jax.experimental.pallas import tpu_sc as plsc

R = 1024
N = 65536
K = 32
NC, NS = 2, 16
NW = NC * NS
RPW = R // NW
CHM = 2048
HALF = N // 2
UNITS = 2 * RPW
VGRP = HALF // 64
GCH = 32768
EPS = 1e-6
INF = float("inf")
IMAX = 2**31 - 1


def _mask_body(d_ref, o_ref, ptx_ref, out_ref):
    vx = ptx_ref[0:1, :] - o_ref[0, 0]
    vy = ptx_ref[1:2, :] - o_ref[0, 1]
    vz = ptx_ref[2:3, :] - o_ref[0, 2]
    vcat = jnp.concatenate([vx, vy, vz], axis=0)
    sq = vcat * vcat
    dsq = jnp.sum(sq, axis=0, keepdims=True)
    db = d_ref[...].astype(jnp.bfloat16)
    vb = vcat.astype(jnp.bfloat16)
    proj = jnp.dot(db, vb, preferred_element_type=jnp.float32)
    perp = jnp.maximum(dsq - proj * proj, 0.0)
    out_ref[...] = jnp.where(proj > 0.0, perp, jnp.inf)


def _masked_tc(d, ray_o, ptx):
    return pl.pallas_call(
        _mask_body,
        grid=(N // CHM,),
        in_specs=[pl.BlockSpec((R, 3), lambda i: (0, 0)),
                  pl.BlockSpec((1, 3), lambda i: (0, 0)),
                  pl.BlockSpec((3, CHM), lambda i: (0, i))],
        out_specs=pl.BlockSpec((R, CHM), lambda i: (0, i)),
        out_shape=jax.ShapeDtypeStruct((R, N), jnp.float32),
    )(d, ray_o, ptx)


def _gat(x, idx):
    return x.at[idx].get(mode="promise_in_bounds")


def _splat_last(x):
    return _gat(x, jnp.full((16,), 15, jnp.int32))


def _vmin_splat(x):
    lane = lax.iota(jnp.int32, 16)
    for m in (8, 4, 2, 1):
        x = jnp.minimum(x, _gat(x, lane ^ m))
    return x


def _merge(ckm, civ, t0k, t0i, t1k, t1i):
    ck, ci = plsc.sort_key_val(ckm, civ)
    rk = lax.rev(ck, (0,))
    ri = lax.rev(ci, (0,))
    c1 = rk < t1k
    mk0 = jnp.where(c1, rk, t1k)
    mi0 = jnp.where(c1, ri, t1i)
    mk, mi = plsc.sort_key_val(mk0, mi0)
    rmk = lax.rev(mk, (0,))
    rmi = lax.rev(mi, (0,))
    c2 = rmk < t0k
    lok = jnp.where(c2, rmk, t0k)
    loi = jnp.where(c2, rmi, t0i)
    hik = jnp.where(c2, t0k, rmk)
    hii = jnp.where(c2, t0i, rmi)
    nt0k, nt0i = plsc.sort_key_val(lok, loi)
    nt1k, nt1i = plsc.sort_key_val(hik, hii)
    return nt0k, nt0i, nt1k, nt1i, _splat_last(nt1k)


def _worker_id():
    return lax.axis_index("s") * NC + lax.axis_index("c")


def _scan_unit(buf, colbase, state):
    def ibody(i, carry):
        t0k, t0i, t1k, t1i, thr = carry
        off = i * 64
        kv = [buf[pl.ds(off + 16 * j, 16)] for j in range(4)]
        oks = [k < thr for k in kv]

        def do(args):
            kv0, kv1, kv2, kv3, off, t0k, t0i, t1k, t1i, thr = args
            kv = (kv0, kv1, kv2, kv3)
            st = (t0k, t0i, t1k, t1i, thr)
            for j in range(4):
                def doj(a, j=j):
                    k, off, t0k, t0i, t1k, t1i, thr = a
                    okj = k < thr
                    ckm = jnp.where(okj, k, INF)
                    civ = lax.iota(jnp.int32, 16) + (colbase + off + 16 * j)
                    return _merge(ckm, civ, t0k, t0i, t1k, t1i)

                def skipj(a):
                    return a[2:]

                okj = kv[j] < st[4]
                st = lax.cond(
                    plsc.all_reduce_population_count(okj)[0] > 0, doj, skipj,
                    (kv[j], off) + st)
            return st

        def skip(args):
            return args[5:]

        anyv = (oks[0] | oks[1]) | (oks[2] | oks[3])
        return lax.cond(
            plsc.all_reduce_population_count(anyv)[0] > 0, do, skip,
            (kv[0], kv[1], kv[2], kv[3], off, t0k, t0i, t1k, t1i, thr))

    return lax.fori_loop(0, VGRP, ibody, state)


def _sc_body(keys_hbm, px_hbm, py_hbm, pz_hbm,
             idx_hbm, perp_hbm, sx_hbm, sy_hbm, sz_hbm,
             bufA, bufB, bufC, stk, sti, okk, oii,
             sxv, syv, szv, semA, semB):
    wid = _worker_id()
    ray_base = wid * RPW
    out_base = wid * (RPW * K)

    def init_body(j, _):
        stk[pl.ds(j * 16, 16)] = jnp.full((16,), INF, jnp.float32)
        sti[pl.ds(j * 16, 16)] = jnp.zeros((16,), jnp.int32)
        return 0
    lax.fori_loop(0, 2 * RPW, init_body, 0)

    def unit_src(u):
        base = (ray_base + (u // 2)) * N + (u % 2) * HALF
        return keys_hbm.at[pl.ds(base, HALF)]

    def load_state(r):
        s0 = r * 32
        return (stk[pl.ds(s0, 16)], sti[pl.ds(s0, 16)],
                stk[pl.ds(s0 + 16, 16)], sti[pl.ds(s0 + 16, 16)])

    def store_state(r, st):
        t0k, t0i, t1k, t1i, _thr = st
        s0 = r * 32
        stk[pl.ds(s0, 16)] = t0k
        sti[pl.ds(s0, 16)] = t0i
        stk[pl.ds(s0 + 16, 16)] = t1k
        sti[pl.ds(s0 + 16, 16)] = t1i
        return 0

    def process(u, buf):
        r = u // 2
        t0k, t0i, t1k, t1i = load_state(r)
        st = (t0k, t0i, t1k, t1i, _splat_last(t1k))
        st = _scan_unit(buf, (u % 2) * HALF, st)
        return store_state(r, st)

    pltpu.async_copy(unit_src(0), bufA, semA)

    def pair_body(p, _):
        u0 = 2 * p
        pltpu.async_copy(unit_src(u0 + 1), bufB, semB)
        pltpu.make_async_copy(unit_src(u0), bufA, semA).wait()
        process(u0, bufA)

        @pl.when(p + 1 < UNITS // 2)
        def _():
            pltpu.async_copy(unit_src(u0 + 2), bufA, semA)
        pltpu.make_async_copy(unit_src(u0 + 1), bufB, semB).wait()
        process(u0 + 1, bufB)
        return 0
    lax.fori_loop(0, UNITS // 2, pair_body, 0)

    lane = lax.iota(jnp.int32, 16)

    def fbody(r, _):
        s0 = r * 32
        k0 = stk[pl.ds(s0, 16)]
        k1 = stk[pl.ds(s0 + 16, 16)]
        i0 = sti[pl.ds(s0, 16)]
        i1 = sti[pl.ds(s0 + 16, 16)]
        ok0 = jnp.zeros((16,), jnp.float32)
        ok1 = jnp.zeros((16,), jnp.float32)
        oi0 = jnp.zeros((16,), jnp.int32)
        oi1 = jnp.zeros((16,), jnp.int32)
        for j in range(K):
            mn = _vmin_splat(jnp.minimum(k0, k1))
            c0 = k0 == mn
            c1 = k1 == mn
            mi = _vmin_splat(jnp.minimum(jnp.where(c0, i0, IMAX),
                                         jnp.where(c1, i1, IMAX)))
            if j < 16:
                sel = lane == j
                ok0 = jnp.where(sel, mn, ok0)
                oi0 = jnp.where(sel, mi, oi0)
            else:
                sel = lane == (j - 16)
                ok1 = jnp.where(sel, mn, ok1)
                oi1 = jnp.where(sel, mi, oi1)
            r0 = c0 & (i0 == mi)
            r1 = c1 & (i1 == mi)
            k0 = jnp.where(r0, INF, k0)
            i0 = jnp.where(r0, IMAX, i0)
            k1 = jnp.where(r1, INF, k1)
            i1 = jnp.where(r1, IMAX, i1)
        okk[pl.ds(s0, 16)] = ok0
        okk[pl.ds(s0 + 16, 16)] = ok1
        oii[pl.ds(s0, 16)] = oi0
        oii[pl.ds(s0 + 16, 16)] = oi1
        return 0
    lax.fori_loop(0, RPW, fbody, 0)

    for c in range(N // GCH):
        sl = pl.ds(c * GCH, GCH)
        pltpu.sync_copy(px_hbm.at[sl], bufA)
        pltpu.sync_copy(py_hbm.at[sl], bufB)
        pltpu.sync_copy(pz_hbm.at[sl], bufC)
        cbase = c * GCH

        def gbody(j, _):
            sl16 = pl.ds(j * 16, 16)
            idxv = oii[sl16]
            inr = (idxv >= cbase) & (idxv < cbase + GCH)
            li = jnp.where(inr, idxv - cbase, 0)
            gx = plsc.load_gather(bufA, [li], mask=inr)
            gy = plsc.load_gather(bufB, [li], mask=inr)
            gz = plsc.load_gather(bufC, [li], mask=inr)
            sxv[sl16] = jnp.where(inr, gx, sxv[sl16])
            syv[sl16] = jnp.where(inr, gy, syv[sl16])
            szv[sl16] = jnp.where(inr, gz, szv[sl16])
            return 0
        lax.fori_loop(0, 2 * RPW, gbody, 0)

    pltpu.sync_copy(oii, idx_hbm.at[pl.ds(out_base, RPW * K)])
    pltpu.sync_copy(okk, perp_hbm.at[pl.ds(out_base, RPW * K)])
    pltpu.sync_copy(sxv, sx_hbm.at[pl.ds(out_base, RPW * K)])
    pltpu.sync_copy(syv, sy_hbm.at[pl.ds(out_base, RPW * K)])
    pltpu.sync_copy(szv, sz_hbm.at[pl.ds(out_base, RPW * K)])


def _sc_search(keys, px, py, pz):
    mesh = plsc.VectorSubcoreMesh(
        core_axis_name="c", subcore_axis_name="s",
        num_cores=NC, num_subcores=NS)
    f32 = jnp.float32
    kern = functools.partial(
        pl.kernel,
        out_type=[jax.ShapeDtypeStruct((R * K,), jnp.int32),
                  jax.ShapeDtypeStruct((R * K,), f32),
                  jax.ShapeDtypeStruct((R * K,), f32),
                  jax.ShapeDtypeStruct((R * K,), f32),
                  jax.ShapeDtypeStruct((R * K,), f32)],
        mesh=mesh,
        compiler_params=pltpu.CompilerParams(needs_layout_passes=False),
        scratch_types=(
            [pltpu.VMEM((HALF,), f32)] * 3
            + [pltpu.VMEM((RPW * K,), f32),
               pltpu.VMEM((RPW * K,), jnp.int32),
               pltpu.VMEM((RPW * K,), f32),
               pltpu.VMEM((RPW * K,), jnp.int32),
               pltpu.VMEM((RPW * K,), f32),
               pltpu.VMEM((RPW * K,), f32),
               pltpu.VMEM((RPW * K,), f32),
               pltpu.SemaphoreType.DMA,
               pltpu.SemaphoreType.DMA]))(_sc_body)
    return kern(keys, px, py, pz)


def _feat_body(sx_ref, sy_ref, sz_ref, perp_ref, d_ref, o_ref,
               dist_ref, projd_ref, az_ref, pitch_ref, sky_ref):
    ox = o_ref[0, 0]
    oy = o_ref[0, 1]
    oz = o_ref[0, 2]
    vx = sx_ref[...] - ox
    vy = sy_ref[...] - oy
    vz = sz_ref[...] - oz
    dist = jnp.sqrt((vx * vx + vy * vy + vz * vz) + EPS)
    dx = d_ref[:, 0:1]
    dy = d_ref[:, 1:2]
    dz = d_ref[:, 2:3]
    dist_ref[...] = dist

    def b16(x):
        return x.astype(jnp.bfloat16).astype(jnp.float32)
    projd_ref[...] = (b16(dx) * b16(vx) + b16(dy) * b16(vy)
                      + b16(dz) * b16(vz))
    az_ref[...] = jnp.arctan2(vy, vx)
    ct = jnp.clip(vz / dist, -1.0 + EPS, 1.0 - EPS)
    st = jnp.sqrt(jnp.maximum((1.0 - ct) * (1.0 + ct), 0.0))
    pitch_ref[...] = jnp.arctan2(st, ct)
    sky_ref[...] = perp_ref[:, 0:1] > 1.0


def _features(sx, sy, sz, perp, d, ray_o):
    f32 = jnp.float32
    return pl.pallas_call(
        _feat_body,
        out_shape=[jax.ShapeDtypeStruct((R, K), f32),
                   jax.ShapeDtypeStruct((R, K), f32),
                   jax.ShapeDtypeStruct((R, K), f32),
                   jax.ShapeDtypeStruct((R, K), f32),
                   jax.ShapeDtypeStruct((R, 1), jnp.bool_)],
    )(sx, sy, sz, perp, d, ray_o)


def kernel(ray_o, ray_d, points):
    d = ray_d / (jnp.linalg.norm(ray_d, axis=-1, keepdims=True) + EPS)
    ray_info = jnp.concatenate(
        [jnp.broadcast_to(ray_o, (R, 3)), d], axis=-1)
    ptx = points.T
    masked = _masked_tc(d, ray_o, ptx)
    keys = jnp.reshape(masked, (R * N,))
    idxf, perpf, sxf, syf, szf = _sc_search(keys, ptx[0], ptx[1], ptx[2])
    idx = idxf.reshape(R, K)
    perp = perpf.reshape(R, K)
    sx = sxf.reshape(R, K)
    sy = syf.reshape(R, K)
    sz = szf.reshape(R, K)
    dist, projd, az, pitch, sky = _features(sx, sy, sz, perp, d, ray_o)
    points_info = jnp.stack([sx, sy, sz, dist, projd, az, pitch], axis=-1)
    return (points_info, ray_info, idx[..., None], sky)
